# Initial kernel scaffold; baseline (speedup 1.0000x reference)
import functools

import jax
import jax.numpy as jnp
from jax import lax
from jax.experimental import pallas as pl
from jax.experimental.pallas import tpu as pltpu
from jax.experimental.pallas import tpu_sc as plsc

EMB_DIM = 32
NUM_CORES = 2
NUM_SUBCORES = 16
NUM_WORKERS = NUM_CORES * NUM_SUBCORES


@functools.lru_cache(maxsize=None)
def _make_gather(B, D, chunk):
    b_per_w = B // NUM_WORKERS
    n_chunks = b_per_w // chunk
    mesh = plsc.VectorSubcoreMesh(core_axis_name="c", subcore_axis_name="s")

    @functools.partial(
        pl.kernel,
        out_type=jax.ShapeDtypeStruct((B, D), jnp.float32),
        mesh=mesh,
        scratch_types=[
            pltpu.VMEM((chunk,), jnp.int32),
            pltpu.VMEM((chunk, D), jnp.float32),
            pltpu.SemaphoreType.DMA,
        ],
    )
    def grab(ids_hbm, table_hbm, out_hbm, idx_v, rows_v, sem):
        wid = lax.axis_index("s") * NUM_CORES + lax.axis_index("c")
        base = wid * b_per_w

        def body(g, carry):
            off = base + g * chunk
            pltpu.sync_copy(ids_hbm.at[pl.ds(off, chunk)], idx_v)
            pltpu.async_copy(table_hbm.at[idx_v], rows_v, sem).wait()
            pltpu.sync_copy(rows_v, out_hbm.at[pl.ds(off, chunk)])
            return carry

        lax.fori_loop(0, n_chunks, body, 0)

    return grab


def kernel(token_ids, emb_weight):
    shape = token_ids.shape
    B = token_ids.size
    D = emb_weight.shape[1]
    ids = token_ids.reshape(B)
    out = _make_gather(B, D, 2000)(ids, emb_weight)
    return out.reshape(*shape, D)


# speedup vs baseline: 1.9681x; 1.9681x over previous
import functools

import jax
import jax.numpy as jnp
from jax import lax
from jax.experimental import pallas as pl
from jax.experimental.pallas import tpu as pltpu
from jax.experimental.pallas import tpu_sc as plsc

EMB_DIM = 32
NUM_CORES = 2
NUM_SUBCORES = 16
NUM_WORKERS = NUM_CORES * NUM_SUBCORES


@functools.lru_cache(maxsize=None)
def _make_gather(B, D, chunk):
    b_per_w = B // NUM_WORKERS
    n_chunks = b_per_w // chunk
    mesh = plsc.VectorSubcoreMesh(core_axis_name="c", subcore_axis_name="s")

    @functools.partial(
        pl.kernel,
        out_type=jax.ShapeDtypeStruct((B, D), jnp.float32),
        mesh=mesh,
        scratch_types=[
            pltpu.VMEM((chunk,), jnp.int32),
            pltpu.VMEM((chunk, D), jnp.float32),
            pltpu.SemaphoreType.DMA,
        ],
        compiler_params=pltpu.CompilerParams(use_tc_tiling_on_sc=False),
    )
    def grab(ids_hbm, table_hbm, out_hbm, idx_v, rows_v, sem):
        wid = lax.axis_index("s") * NUM_CORES + lax.axis_index("c")
        base = wid * b_per_w

        def body(g, carry):
            off = base + g * chunk
            pltpu.sync_copy(ids_hbm.at[pl.ds(off, chunk)], idx_v)
            pltpu.async_copy(table_hbm.at[idx_v], rows_v, sem).wait()
            pltpu.sync_copy(rows_v, out_hbm.at[pl.ds(off, chunk)])
            return carry

        lax.fori_loop(0, n_chunks, body, 0)

    return grab


def kernel(token_ids, emb_weight):
    shape = token_ids.shape
    B = token_ids.size
    D = emb_weight.shape[1]
    ids = token_ids.reshape(B)
    out = _make_gather(B, D, 2000)(ids, emb_weight)
    return out.reshape(*shape, D)


# speedup vs baseline: 1.9802x; 1.0062x over previous
import functools

import jax
import jax.numpy as jnp
from jax import lax
from jax.experimental import pallas as pl
from jax.experimental.pallas import tpu as pltpu
from jax.experimental.pallas import tpu_sc as plsc

EMB_DIM = 32
NUM_CORES = 2
NUM_SUBCORES = 16
NUM_WORKERS = NUM_CORES * NUM_SUBCORES


@functools.lru_cache(maxsize=None)
def _make_gather(B, D, chunk):
    b_per_w = B // NUM_WORKERS
    n = b_per_w // chunk
    mesh = plsc.VectorSubcoreMesh(core_axis_name="c", subcore_axis_name="s")

    @functools.partial(
        pl.kernel,
        out_type=jax.ShapeDtypeStruct((B, D), jnp.float32),
        mesh=mesh,
        scratch_types=[
            pltpu.VMEM((chunk,), jnp.int32),
            pltpu.VMEM((chunk,), jnp.int32),
            pltpu.VMEM((chunk, D), jnp.float32),
            pltpu.VMEM((chunk, D), jnp.float32),
            pltpu.SemaphoreType.DMA,
            pltpu.SemaphoreType.DMA,
            pltpu.SemaphoreType.DMA,
            pltpu.SemaphoreType.DMA,
            pltpu.SemaphoreType.DMA,
            pltpu.SemaphoreType.DMA,
        ],
        compiler_params=pltpu.CompilerParams(use_tc_tiling_on_sc=False),
    )
    def grab(ids_hbm, table_hbm, out_hbm, idx0, idx1, rows0, rows1,
             si0, si1, sg0, sg1, ss0, ss1):
        wid = lax.axis_index("s") * NUM_CORES + lax.axis_index("c")
        base = wid * b_per_w
        idx = (idx0, idx1)
        rows = (rows0, rows1)
        sem_i = (si0, si1)
        sem_g = (sg0, sg1)
        sem_s = (ss0, ss1)

        def idx_start(g):
            off = base + g * chunk
            return pltpu.async_copy(
                ids_hbm.at[pl.ds(off, chunk)], idx[g % 2], sem_i[g % 2])

        def gather_start(g):
            return pltpu.async_copy(
                table_hbm.at[idx[g % 2]], rows[g % 2], sem_g[g % 2])

        def store_start(g):
            off = base + g * chunk
            return pltpu.async_copy(
                rows[g % 2], out_hbm.at[pl.ds(off, chunk)], sem_s[g % 2])

        idx_dma = {0: idx_start(0)}
        if n > 1:
            idx_dma[1] = idx_start(1)
        idx_dma[0].wait()
        gather_dma = gather_start(0)
        store_dma = {}
        for g in range(n):
            gather_dma.wait()
            if g + 2 < n:
                idx_dma[g + 2] = idx_start(g + 2)
            store_dma[g] = store_start(g)
            if g + 1 < n:
                if g >= 1:
                    store_dma[g - 1].wait()
                idx_dma[g + 1].wait()
                gather_dma = gather_start(g + 1)
        if n >= 2:
            store_dma[n - 2].wait()
        store_dma[n - 1].wait()

    return grab


def kernel(token_ids, emb_weight):
    shape = token_ids.shape
    B = token_ids.size
    D = emb_weight.shape[1]
    ids = token_ids.reshape(B)
    out = _make_gather(B, D, 1600)(ids, emb_weight)
    return out.reshape(*shape, D)


# speedup vs baseline: 2.0643x; 1.0425x over previous
import functools

import jax
import jax.numpy as jnp
from jax import lax
from jax.experimental import pallas as pl
from jax.experimental.pallas import tpu as pltpu
from jax.experimental.pallas import tpu_sc as plsc

EMB_DIM = 32
NUM_CORES = 2
NUM_SUBCORES = 16
NUM_WORKERS = NUM_CORES * NUM_SUBCORES


@functools.lru_cache(maxsize=None)
def _make_gather(B, D, chunk):
    b_per_w = B // NUM_WORKERS
    n = b_per_w // chunk
    mesh = plsc.VectorSubcoreMesh(core_axis_name="c", subcore_axis_name="s")

    @functools.partial(
        pl.kernel,
        out_type=jax.ShapeDtypeStruct((B, D), jnp.float32),
        mesh=mesh,
        scratch_types=[
            pltpu.VMEM((chunk,), jnp.int32),
            pltpu.VMEM((chunk,), jnp.int32),
            pltpu.VMEM((chunk, D), jnp.float32),
            pltpu.VMEM((chunk, D), jnp.float32),
            pltpu.SemaphoreType.DMA,
            pltpu.SemaphoreType.DMA,
            pltpu.SemaphoreType.DMA,
            pltpu.SemaphoreType.DMA,
            pltpu.SemaphoreType.DMA,
            pltpu.SemaphoreType.DMA,
        ],
        compiler_params=pltpu.CompilerParams(use_tc_tiling_on_sc=False),
    )
    def grab(ids_hbm, table_hbm, out_hbm, idx0, idx1, rows0, rows1,
             si0, si1, sg0, sg1, ss0, ss1):
        wid = lax.axis_index("s") * NUM_CORES + lax.axis_index("c")
        base = wid * b_per_w
        idx = (idx0, idx1)
        rows = (rows0, rows1)
        sem_i = (si0, si1)
        sem_g = (sg0, sg1)
        sem_s = (ss0, ss1)

        def idx_start(g):
            off = base + g * chunk
            return pltpu.async_copy(
                ids_hbm.at[pl.ds(off, chunk)], idx[g % 2], sem_i[g % 2])

        def gather_start(g):
            return pltpu.async_copy(
                table_hbm.at[idx[g % 2]], rows[g % 2], sem_g[g % 2])

        def store_start(g):
            off = base + g * chunk
            return pltpu.async_copy(
                rows[g % 2], out_hbm.at[pl.ds(off, chunk)], sem_s[g % 2])

        idx_dma = {0: idx_start(0)}
        if n > 1:
            idx_dma[1] = idx_start(1)
        idx_dma[0].wait()
        gather_dma = gather_start(0)
        store_dma = {}
        for g in range(n):
            gather_dma.wait()
            if g + 2 < n:
                idx_dma[g + 2] = idx_start(g + 2)
            store_dma[g] = store_start(g)
            if g + 1 < n:
                if g >= 1:
                    store_dma[g - 1].wait()
                idx_dma[g + 1].wait()
                gather_dma = gather_start(g + 1)
        if n >= 2:
            store_dma[n - 2].wait()
        store_dma[n - 1].wait()

    return grab


def kernel(token_ids, emb_weight):
    b, t, w = token_ids.shape
    B = token_ids.size
    D = emb_weight.shape[1]
    ids = token_ids.reshape(B)
    rows = _make_gather(B, D, 1600)(ids, emb_weight)
    r3 = rows.reshape(b, t * w, D)
    out = jnp.transpose(r3, (1, 2, 0))
    return jnp.transpose(out.reshape(t, w, D, b), (3, 0, 1, 2))


# speedup vs baseline: 2.0743x; 1.0049x over previous
import functools

import jax
import jax.numpy as jnp
from jax import lax
from jax.experimental import pallas as pl
from jax.experimental.pallas import tpu as pltpu
from jax.experimental.pallas import tpu_sc as plsc

EMB_DIM = 32
NUM_CORES = 2
NUM_SUBCORES = 16
NUM_WORKERS = NUM_CORES * NUM_SUBCORES


@functools.lru_cache(maxsize=None)
def _make_gather(B, D, chunk):
    b_per_w = B // NUM_WORKERS
    n = b_per_w // chunk
    mesh = plsc.VectorSubcoreMesh(core_axis_name="c", subcore_axis_name="s")

    @functools.partial(
        pl.kernel,
        out_type=jax.ShapeDtypeStruct((B, D), jnp.float32),
        mesh=mesh,
        scratch_types=[
            pltpu.VMEM((chunk,), jnp.int32),
            pltpu.VMEM((chunk,), jnp.int32),
            pltpu.VMEM((chunk,), jnp.int32),
            pltpu.VMEM((chunk, D), jnp.float32),
            pltpu.VMEM((chunk, D), jnp.float32),
            pltpu.VMEM((chunk, D), jnp.float32),
            pltpu.SemaphoreType.DMA,
            pltpu.SemaphoreType.DMA,
            pltpu.SemaphoreType.DMA,
            pltpu.SemaphoreType.DMA,
            pltpu.SemaphoreType.DMA,
            pltpu.SemaphoreType.DMA,
            pltpu.SemaphoreType.DMA,
            pltpu.SemaphoreType.DMA,
            pltpu.SemaphoreType.DMA,
        ],
        compiler_params=pltpu.CompilerParams(use_tc_tiling_on_sc=False),
    )
    def grab(ids_hbm, table_hbm, out_hbm, idx0, idx1, idx2,
             rows0, rows1, rows2, si0, si1, si2, sg0, sg1, sg2,
             ss0, ss1, ss2):
        wid = lax.axis_index("s") * NUM_CORES + lax.axis_index("c")
        base = wid * b_per_w
        idx = (idx0, idx1, idx2)
        rows = (rows0, rows1, rows2)
        sem_i = (si0, si1, si2)
        sem_g = (sg0, sg1, sg2)
        sem_s = (ss0, ss1, ss2)

        def idx_start(g):
            off = base + g * chunk
            return pltpu.async_copy(
                ids_hbm.at[pl.ds(off, chunk)], idx[g % 3], sem_i[g % 3])

        def gather_start(g):
            return pltpu.async_copy(
                table_hbm.at[idx[g % 3]], rows[g % 3], sem_g[g % 3])

        def store_start(g):
            off = base + g * chunk
            return pltpu.async_copy(
                rows[g % 3], out_hbm.at[pl.ds(off, chunk)], sem_s[g % 3])

        idx_dma = {}
        for g in range(min(3, n)):
            idx_dma[g] = idx_start(g)
        gather_dma = {}
        idx_dma[0].wait()
        gather_dma[0] = gather_start(0)
        if n > 1:
            idx_dma[1].wait()
            gather_dma[1] = gather_start(1)
        store_dma = {}
        for g in range(n):
            gather_dma[g].wait()
            if g + 3 < n:
                idx_dma[g + 3] = idx_start(g + 3)
            store_dma[g] = store_start(g)
            if g + 2 < n:
                if g >= 1:
                    store_dma[g - 1].wait()
                idx_dma[g + 2].wait()
                gather_dma[g + 2] = gather_start(g + 2)
        for g in range(max(0, n - 2), n):
            store_dma[g].wait()

    return grab


def kernel(token_ids, emb_weight):
    b, t, w = token_ids.shape
    B = token_ids.size
    D = emb_weight.shape[1]
    ids = token_ids.reshape(B)
    rows = _make_gather(B, D, 1280)(ids, emb_weight)
    r3 = rows.reshape(b, t * w, D)
    out = jnp.transpose(r3, (1, 2, 0))
    return jnp.transpose(out.reshape(t, w, D, b), (3, 0, 1, 2))
